# baseline (speedup 1.0000x reference)
import functools

import jax
import jax.numpy as jnp
from jax import lax
from jax.experimental import pallas as pl
from jax.experimental.pallas import tpu as pltpu
from jax.experimental.pallas import tpu_sc as plsc

N = 10000
E = 320000
D_IN = 128
NHID = 64
NLAT = 16

NC = 2
NS = 16
L = 16
NW = NC * NS
CHUNK = 128
CH_PER_W = 80
EPW = CH_PER_W * CHUNK
EP = NW * EPW
NP = 10112
ROWS_PER_TILE = NP // NS

_MESH = dict(core_axis_name="c", subcore_axis_name="s")


def _deg_body(dst_hbm, out_hbm, idx_v, deg_v):
  cid = lax.axis_index("c")
  sid = lax.axis_index("s")
  w = cid * NS + sid
  pltpu.sync_copy(dst_hbm.at[pl.ds(w * EPW, EPW)], idx_v)

  @pl.loop(0, NP // L)
  def _(i):
    deg_v[pl.ds(i * L, L)] = jnp.zeros((L,), jnp.float32)

  ones = jnp.ones((L,), jnp.float32)

  @pl.loop(0, EPW // L)
  def _(j):
    idx = idx_v[pl.ds(j * L, L)]
    plsc.addupdate_scatter(deg_v, [idx], ones)

  pltpu.sync_copy(deg_v, out_hbm.at[w])


def _deg(dstp):
  mesh = plsc.VectorSubcoreMesh(**_MESH)
  return pl.kernel(
      _deg_body,
      out_type=jax.ShapeDtypeStruct((NW, NP), jnp.float32),
      mesh=mesh,
      scratch_types=[
          pltpu.VMEM((EPW,), jnp.int32),
          pltpu.VMEM((NP,), jnp.float32),
      ],
      compiler_params=pltpu.CompilerParams(needs_layout_passes=False),
  )(dstp)


def _agg_body(depth, hp_hbm, srcm_hbm, dstm_hbm, zeros_hbm, out_hbm,
              src2_v, dst2_v, rows_v, table_sh, acc_sh, sg, ss):
  cid = lax.axis_index("c")
  sid = lax.axis_index("s")
  w = cid * NS + sid
  r0 = sid * ROWS_PER_TILE
  tbl_cp = pltpu.async_copy(hp_hbm.at[pl.ds(r0, ROWS_PER_TILE)],
                            table_sh.at[pl.ds(r0, ROWS_PER_TILE)], sg.at[0])
  zro_cp = pltpu.async_copy(zeros_hbm.at[pl.ds(r0, ROWS_PER_TILE)],
                            acc_sh.at[pl.ds(r0, ROWS_PER_TILE)], sg.at[1])
  src_cp = pltpu.async_copy(srcm_hbm.at[pl.ds(w * CH_PER_W, CH_PER_W)],
                            src2_v, ss.at[0])
  dst_cp = pltpu.async_copy(dstm_hbm.at[pl.ds(w * CH_PER_W, CH_PER_W)],
                            dst2_v, ss.at[1])
  tbl_cp.wait()
  zro_cp.wait()
  src_cp.wait()
  dst_cp.wait()
  plsc.subcore_barrier()

  def gather(c, i):
    pltpu.async_copy(table_sh.at[src2_v.at[c]], rows_v.at[i], sg.at[i])

  def wait_gather(c, i):
    pltpu.make_async_copy(table_sh.at[src2_v.at[c]], rows_v.at[i],
                          sg.at[i]).wait()

  def scatter(c, i):
    pltpu.async_copy(rows_v.at[i], acc_sh.at[dst2_v.at[c]], ss.at[i], add=True)

  def wait_scatter(c, i):
    pltpu.make_async_copy(rows_v.at[i], acc_sh.at[dst2_v.at[c]],
                          ss.at[i]).wait()

  nfull = (CH_PER_W // depth) * depth
  tail = CH_PER_W - nfull

  for i in range(depth):
    gather(i, i)

  @pl.loop(0, nfull, step=depth)
  def _(k):
    for i in range(depth):
      wait_gather(k + i, i)
      scatter(k + i, i)
    for i in range(depth):
      wait_scatter(k + i, i)
      knext = jnp.where(k + depth + i >= CH_PER_W, i, k + depth + i)
      gather(knext, i)

  for i in range(tail):
    wait_gather(nfull + i, i)
    scatter(nfull + i, i)
  for i in range(tail):
    wait_scatter(nfull + i, i)
  for i in range(tail, depth):
    wait_gather(i, i)

  plsc.subcore_barrier()
  pltpu.sync_copy(acc_sh.at[pl.ds(r0, ROWS_PER_TILE)],
                  out_hbm.at[cid, pl.ds(r0, ROWS_PER_TILE)])


def _agg(hp, srcm, dstm, zeros_np, f, depth):
  mesh = plsc.VectorSubcoreMesh(**_MESH)
  return pl.kernel(
      functools.partial(_agg_body, depth),
      out_type=jax.ShapeDtypeStruct((NC, NP, f), jnp.float32),
      mesh=mesh,
      scratch_types=[
          pltpu.VMEM((CH_PER_W, CHUNK), jnp.int32),
          pltpu.VMEM((CH_PER_W, CHUNK), jnp.int32),
          pltpu.VMEM((depth, CHUNK, f), jnp.float32),
          pltpu.VMEM_SHARED((NP, f), jnp.float32),
          pltpu.VMEM_SHARED((NP, f), jnp.float32),
          pltpu.SemaphoreType.DMA((depth,)),
          pltpu.SemaphoreType.DMA((depth,)),
      ],
      compiler_params=pltpu.CompilerParams(use_tc_tiling_on_sc=False),
  )(hp, srcm, dstm, zeros_np)


def _mm1_body(x_ref, w1_ref, h1_ref):
  h1_ref[...] = jnp.dot(x_ref[...], w1_ref[...],
                        preferred_element_type=jnp.float32,
                        precision=lax.Precision.HIGHEST)


def _mm1(x, W1):
  bm = 1000
  return pl.pallas_call(
      _mm1_body,
      grid=(N // bm,),
      in_specs=[
          pl.BlockSpec((bm, D_IN), lambda i: (i, 0)),
          pl.BlockSpec((D_IN, NHID), lambda i: (0, 0)),
      ],
      out_specs=pl.BlockSpec((bm, NHID), lambda i: (i, 0)),
      out_shape=jax.ShapeDtypeStruct((NP, NHID), jnp.float32),
  )(x, W1)


def _dinv_body(degp_ref, h1_ref, dinv_ref, h1p_ref):
  deg = jnp.sum(degp_ref[...], axis=0) + 1.0
  dinv = lax.rsqrt(deg)[:, None]
  dinv_ref[...] = dinv
  h1p_ref[...] = h1_ref[...] * dinv


def _dinv(degp, h1):
  return pl.pallas_call(
      _dinv_body,
      grid=(1,),
      in_specs=[
          pl.BlockSpec((NW, NP), lambda i: (0, 0)),
          pl.BlockSpec((NP, NHID), lambda i: (0, 0)),
      ],
      out_specs=[
          pl.BlockSpec((NP, 1), lambda i: (0, 0)),
          pl.BlockSpec((NP, NHID), lambda i: (0, 0)),
      ],
      out_shape=[
          jax.ShapeDtypeStruct((NP, 1), jnp.float32),
          jax.ShapeDtypeStruct((NP, NHID), jnp.float32),
      ],
  )(degp, h1)


def _mid_body(p_ref, h1p_ref, dinv_ref, b1_ref, w2_ref, h2p_ref):
  dinv = dinv_ref[...]
  s = (p_ref[0] + p_ref[1] + h1p_ref[...]) * dinv + b1_ref[...]
  h = jnp.maximum(s, 0.0)
  h2 = jnp.dot(h, w2_ref[...], preferred_element_type=jnp.float32,
               precision=lax.Precision.HIGHEST)
  h2p_ref[...] = h2 * dinv


def _mid(p, h1p, dinv, b1, W2):
  bm = 1000
  return pl.pallas_call(
      _mid_body,
      grid=(N // bm,),
      in_specs=[
          pl.BlockSpec((NC, bm, NHID), lambda i: (0, i, 0)),
          pl.BlockSpec((bm, NHID), lambda i: (i, 0)),
          pl.BlockSpec((bm, 1), lambda i: (i, 0)),
          pl.BlockSpec((1, NHID), lambda i: (0, 0)),
          pl.BlockSpec((NHID, NLAT), lambda i: (0, 0)),
      ],
      out_specs=pl.BlockSpec((bm, NLAT), lambda i: (i, 0)),
      out_shape=jax.ShapeDtypeStruct((NP, NLAT), jnp.float32),
  )(p, h1p, dinv, b1, W2)


def _zk_body(q_ref, h2p_ref, dinv_ref, b2_ref, z_ref):
  z = (q_ref[0] + q_ref[1] + h2p_ref[...]) * dinv_ref[...] + b2_ref[...]
  z_ref[...] = z.astype(jnp.bfloat16)


def _zk(q, h2p, dinv, b2):
  bm = 1000
  return pl.pallas_call(
      _zk_body,
      grid=(N // bm,),
      in_specs=[
          pl.BlockSpec((NC, bm, NLAT), lambda i: (0, i, 0)),
          pl.BlockSpec((bm, NLAT), lambda i: (i, 0)),
          pl.BlockSpec((bm, 1), lambda i: (i, 0)),
          pl.BlockSpec((1, NLAT), lambda i: (0, 0)),
      ],
      out_specs=pl.BlockSpec((bm, NLAT), lambda i: (i, 0)),
      out_shape=jax.ShapeDtypeStruct((N, NLAT), jnp.bfloat16),
  )(q, h2p, dinv, b2)


def _dec_body(zi_ref, zj_ref, out_ref):
  out_ref[...] = lax.dot_general(
      zi_ref[...], zj_ref[...],
      dimension_numbers=(((1,), (1,)), ((), ())),
      preferred_element_type=jnp.float32)


def _dec(z):
  bm = 400
  return pl.pallas_call(
      _dec_body,
      grid=(N // bm,),
      in_specs=[
          pl.BlockSpec((bm, NLAT), lambda i: (i, 0)),
          pl.BlockSpec((N, NLAT), lambda i: (0, 0)),
      ],
      out_specs=pl.BlockSpec((bm, N), lambda i: (i, 0)),
      out_shape=jax.ShapeDtypeStruct((N, N), jnp.float32),
      compiler_params=pltpu.CompilerParams(
          dimension_semantics=("parallel",)),
  )(z, z)


def kernel(x, edge_index, W1, b1, W2, b2):
  src = edge_index[0]
  dst = edge_index[1]
  pad = EP - E
  srcp = jnp.concatenate([src, jnp.zeros((pad,), jnp.int32)])
  dstp = jnp.concatenate([dst, jnp.full((pad,), N, jnp.int32)])
  srcm = srcp.reshape(NW * CH_PER_W, CHUNK)
  dstm = dstp.reshape(NW * CH_PER_W, CHUNK)
  zeros64 = jnp.zeros((NP, NHID), jnp.float32)
  zeros16 = jnp.zeros((NP, NLAT), jnp.float32)

  degp = _deg(dstp)
  h1 = _mm1(x, W1)
  dinv, h1p = _dinv(degp, h1)
  p = _agg(h1p, srcm, dstm, zeros64, NHID, 3)
  h2p = _mid(p, h1p, dinv, b1.reshape(1, NHID), W2)
  q = _agg(h2p, srcm, dstm, zeros16, NLAT, 4)
  z = _zk(q, h2p, dinv, b2.reshape(1, NLAT))
  return _dec(z)


# speedup vs baseline: 1.0341x; 1.0341x over previous
import functools

import jax
import jax.numpy as jnp
from jax import lax
from jax.experimental import pallas as pl
from jax.experimental.pallas import tpu as pltpu
from jax.experimental.pallas import tpu_sc as plsc

N = 10000
E = 320000
D_IN = 128
NHID = 64
NLAT = 16

NC = 2
NS = 16
L = 16
NW = NC * NS
CHUNK = 128
CH_PER_W = 80
EPW = CH_PER_W * CHUNK
EP = NW * EPW
NP = 10112
ROWS_PER_TILE = NP // NS

_MESH = dict(core_axis_name="c", subcore_axis_name="s")


def _deg_body(dst_hbm, out_hbm, idx_v, deg_v):
  cid = lax.axis_index("c")
  sid = lax.axis_index("s")
  w = cid * NS + sid
  pltpu.sync_copy(dst_hbm.at[pl.ds(w * EPW, EPW)], idx_v)

  @pl.loop(0, NP // L)
  def _(i):
    deg_v[pl.ds(i * L, L)] = jnp.zeros((L,), jnp.float32)

  ones = jnp.ones((L,), jnp.float32)

  @pl.loop(0, EPW // L)
  def _(j):
    idx = idx_v[pl.ds(j * L, L)]
    plsc.addupdate_scatter(deg_v, [idx], ones)

  pltpu.sync_copy(deg_v, out_hbm.at[w])


def _deg(dstp):
  mesh = plsc.VectorSubcoreMesh(**_MESH)
  return pl.kernel(
      _deg_body,
      out_type=jax.ShapeDtypeStruct((NW, NP), jnp.float32),
      mesh=mesh,
      scratch_types=[
          pltpu.VMEM((EPW,), jnp.int32),
          pltpu.VMEM((NP,), jnp.float32),
      ],
      compiler_params=pltpu.CompilerParams(needs_layout_passes=False),
  )(dstp)


def _agg_body(depth, hp_hbm, srcm_hbm, dstm_hbm, zeros_hbm, out_hbm,
              src2_v, dst2_v, rows_v, table_sh, acc_sh, sg, ss):
  cid = lax.axis_index("c")
  sid = lax.axis_index("s")
  w = cid * NS + sid
  r0 = sid * ROWS_PER_TILE
  tbl_cp = pltpu.async_copy(hp_hbm.at[pl.ds(r0, ROWS_PER_TILE)],
                            table_sh.at[pl.ds(r0, ROWS_PER_TILE)], sg.at[0])
  zro_cp = pltpu.async_copy(zeros_hbm.at[pl.ds(r0, ROWS_PER_TILE)],
                            acc_sh.at[pl.ds(r0, ROWS_PER_TILE)], sg.at[1])
  src_cp = pltpu.async_copy(srcm_hbm.at[pl.ds(w * CH_PER_W, CH_PER_W)],
                            src2_v, ss.at[0])
  dst_cp = pltpu.async_copy(dstm_hbm.at[pl.ds(w * CH_PER_W, CH_PER_W)],
                            dst2_v, ss.at[1])
  tbl_cp.wait()
  zro_cp.wait()
  src_cp.wait()
  dst_cp.wait()
  plsc.subcore_barrier()

  def gather(c, i):
    pltpu.async_copy(table_sh.at[src2_v.at[c]], rows_v.at[i], sg.at[i])

  def wait_gather(c, i):
    pltpu.make_async_copy(table_sh.at[src2_v.at[c]], rows_v.at[i],
                          sg.at[i]).wait()

  def scatter(c, i):
    pltpu.async_copy(rows_v.at[i], acc_sh.at[dst2_v.at[c]], ss.at[i], add=True)

  def wait_scatter(c, i):
    pltpu.make_async_copy(rows_v.at[i], acc_sh.at[dst2_v.at[c]],
                          ss.at[i]).wait()

  nfull = (CH_PER_W // depth) * depth
  tail = CH_PER_W - nfull

  for i in range(depth):
    gather(i, i)

  @pl.loop(0, nfull, step=depth)
  def _(k):
    for i in range(depth):
      wait_gather(k + i, i)
      scatter(k + i, i)
    for i in range(depth):
      wait_scatter(k + i, i)
      knext = jnp.where(k + depth + i >= CH_PER_W, i, k + depth + i)
      gather(knext, i)

  for i in range(tail):
    wait_gather(nfull + i, i)
    scatter(nfull + i, i)
  for i in range(tail):
    wait_scatter(nfull + i, i)
  for i in range(tail, depth):
    wait_gather(i, i)

  plsc.subcore_barrier()
  pltpu.sync_copy(acc_sh.at[pl.ds(r0, ROWS_PER_TILE)],
                  out_hbm.at[cid, pl.ds(r0, ROWS_PER_TILE)])


def _agg(hp, srcm, dstm, zeros_np, f, depth):
  mesh = plsc.VectorSubcoreMesh(**_MESH)
  return pl.kernel(
      functools.partial(_agg_body, depth),
      out_type=jax.ShapeDtypeStruct((NC, NP, f), jnp.float32),
      mesh=mesh,
      scratch_types=[
          pltpu.VMEM((CH_PER_W, CHUNK), jnp.int32),
          pltpu.VMEM((CH_PER_W, CHUNK), jnp.int32),
          pltpu.VMEM((depth, CHUNK, f), jnp.float32),
          pltpu.VMEM_SHARED((NP, f), jnp.float32),
          pltpu.VMEM_SHARED((NP, f), jnp.float32),
          pltpu.SemaphoreType.DMA((depth,)),
          pltpu.SemaphoreType.DMA((depth,)),
      ],
      compiler_params=pltpu.CompilerParams(use_tc_tiling_on_sc=False),
  )(hp, srcm, dstm, zeros_np)


def _mm1_body(x_ref, w1_ref, h1_ref):
  h1_ref[...] = jnp.dot(x_ref[...], w1_ref[...],
                        preferred_element_type=jnp.float32,
                        precision=lax.Precision.HIGHEST)


def _mm1(x, W1):
  bm = 1000
  return pl.pallas_call(
      _mm1_body,
      grid=(N // bm,),
      in_specs=[
          pl.BlockSpec((bm, D_IN), lambda i: (i, 0)),
          pl.BlockSpec((D_IN, NHID), lambda i: (0, 0)),
      ],
      out_specs=pl.BlockSpec((bm, NHID), lambda i: (i, 0)),
      out_shape=jax.ShapeDtypeStruct((NP, NHID), jnp.float32),
  )(x, W1)


def _dinv_body(degp_ref, h1_ref, dinv_ref, h1p_ref):
  deg = jnp.sum(degp_ref[...], axis=0) + 1.0
  dinv = lax.rsqrt(deg)[:, None]
  dinv_ref[...] = dinv
  h1p_ref[...] = h1_ref[...] * dinv


def _dinv(degp, h1):
  return pl.pallas_call(
      _dinv_body,
      grid=(1,),
      in_specs=[
          pl.BlockSpec((NW, NP), lambda i: (0, 0)),
          pl.BlockSpec((NP, NHID), lambda i: (0, 0)),
      ],
      out_specs=[
          pl.BlockSpec((NP, 1), lambda i: (0, 0)),
          pl.BlockSpec((NP, NHID), lambda i: (0, 0)),
      ],
      out_shape=[
          jax.ShapeDtypeStruct((NP, 1), jnp.float32),
          jax.ShapeDtypeStruct((NP, NHID), jnp.float32),
      ],
  )(degp, h1)


def _mid_body(p_ref, h1p_ref, dinv_ref, b1_ref, w2_ref, h2p_ref):
  dinv = dinv_ref[...]
  s = (p_ref[0] + p_ref[1] + h1p_ref[...]) * dinv + b1_ref[...]
  h = jnp.maximum(s, 0.0)
  h2 = jnp.dot(h, w2_ref[...], preferred_element_type=jnp.float32,
               precision=lax.Precision.HIGHEST)
  h2p_ref[...] = h2 * dinv


def _mid(p, h1p, dinv, b1, W2):
  bm = 1000
  return pl.pallas_call(
      _mid_body,
      grid=(N // bm,),
      in_specs=[
          pl.BlockSpec((NC, bm, NHID), lambda i: (0, i, 0)),
          pl.BlockSpec((bm, NHID), lambda i: (i, 0)),
          pl.BlockSpec((bm, 1), lambda i: (i, 0)),
          pl.BlockSpec((1, NHID), lambda i: (0, 0)),
          pl.BlockSpec((NHID, NLAT), lambda i: (0, 0)),
      ],
      out_specs=pl.BlockSpec((bm, NLAT), lambda i: (i, 0)),
      out_shape=jax.ShapeDtypeStruct((NP, NLAT), jnp.float32),
  )(p, h1p, dinv, b1, W2)


def _zk_body(q_ref, h2p_ref, dinv_ref, b2_ref, z_ref):
  z = (q_ref[0] + q_ref[1] + h2p_ref[...]) * dinv_ref[...] + b2_ref[...]
  z_ref[...] = z.astype(jnp.bfloat16)


def _zk(q, h2p, dinv, b2):
  bm = 1000
  return pl.pallas_call(
      _zk_body,
      grid=(N // bm,),
      in_specs=[
          pl.BlockSpec((NC, bm, NLAT), lambda i: (0, i, 0)),
          pl.BlockSpec((bm, NLAT), lambda i: (i, 0)),
          pl.BlockSpec((bm, 1), lambda i: (i, 0)),
          pl.BlockSpec((1, NLAT), lambda i: (0, 0)),
      ],
      out_specs=pl.BlockSpec((bm, NLAT), lambda i: (i, 0)),
      out_shape=jax.ShapeDtypeStruct((N, NLAT), jnp.bfloat16),
  )(q, h2p, dinv, b2)


def _dec_body(zi_ref, zj_ref, out_ref):
  out_ref[...] = lax.dot_general(
      zi_ref[...], zj_ref[...],
      dimension_numbers=(((1,), (1,)), ((), ())),
      preferred_element_type=jnp.float32)


def _dec(z):
  bm = 400
  return pl.pallas_call(
      _dec_body,
      grid=(N // bm,),
      in_specs=[
          pl.BlockSpec((bm, NLAT), lambda i: (i, 0)),
          pl.BlockSpec((N, NLAT), lambda i: (0, 0)),
      ],
      out_specs=pl.BlockSpec((bm, N), lambda i: (i, 0)),
      out_shape=jax.ShapeDtypeStruct((N, N), jnp.float32),
      compiler_params=pltpu.CompilerParams(
          dimension_semantics=("parallel",)),
  )(z, z)


def kernel(x, edge_index, W1, b1, W2, b2):
  src = edge_index[0]
  dst = edge_index[1]
  pad = EP - E
  srcp = jnp.concatenate([src, jnp.zeros((pad,), jnp.int32)])
  dstp = jnp.concatenate([dst, jnp.full((pad,), N, jnp.int32)])
  srcm = srcp.reshape(NW * CH_PER_W, CHUNK)
  dstm = dstp.reshape(NW * CH_PER_W, CHUNK)
  zeros64 = jnp.zeros((NP, NHID), jnp.float32)
  zeros16 = jnp.zeros((NP, NLAT), jnp.float32)

  degp = _deg(dstp)
  h1 = _mm1(x, W1)
  dinv, h1p = _dinv(degp, h1)
  p = _agg(h1p, srcm, dstm, zeros64, NHID, 2)
  h2p = _mid(p, h1p, dinv, b1.reshape(1, NHID), W2)
  q = _agg(h2p, srcm, dstm, zeros16, NLAT, 4)
  z = _zk(q, h2p, dinv, b2.reshape(1, NLAT))
  return _dec(z)


# speedup vs baseline: 1.0359x; 1.0017x over previous
import functools

import jax
import jax.numpy as jnp
from jax import lax
from jax.experimental import pallas as pl
from jax.experimental.pallas import tpu as pltpu
from jax.experimental.pallas import tpu_sc as plsc

N = 10000
E = 320000
D_IN = 128
NHID = 64
NLAT = 16

NC = 2
NS = 16
L = 16
NW = NC * NS
CHUNK = 128
CH_PER_W = 80
EPW = CH_PER_W * CHUNK
EP = NW * EPW
NP = 10112
ROWS_PER_TILE = NP // NS

_MESH = dict(core_axis_name="c", subcore_axis_name="s")


def _deg_body(dst_hbm, out_hbm, idx_v, deg_v):
  cid = lax.axis_index("c")
  sid = lax.axis_index("s")
  w = cid * NS + sid
  pltpu.sync_copy(dst_hbm.at[pl.ds(w * EPW, EPW)], idx_v)

  @pl.loop(0, NP // L)
  def _(i):
    deg_v[pl.ds(i * L, L)] = jnp.zeros((L,), jnp.float32)

  ones = jnp.ones((L,), jnp.float32)

  @pl.loop(0, EPW // L)
  def _(j):
    idx = idx_v[pl.ds(j * L, L)]
    plsc.addupdate_scatter(deg_v, [idx], ones)

  pltpu.sync_copy(deg_v, out_hbm.at[w])


def _deg(dstp):
  mesh = plsc.VectorSubcoreMesh(**_MESH)
  return pl.kernel(
      _deg_body,
      out_type=jax.ShapeDtypeStruct((NW, NP), jnp.float32),
      mesh=mesh,
      scratch_types=[
          pltpu.VMEM((EPW,), jnp.int32),
          pltpu.VMEM((NP,), jnp.float32),
      ],
      compiler_params=pltpu.CompilerParams(needs_layout_passes=False),
  )(dstp)


def _agg_body(depth, hp_hbm, srcm_hbm, dstm_hbm, zeros_hbm, out_hbm,
              src2_v, dst2_v, rows_v, table_sh, acc_sh, sg, ss):
  cid = lax.axis_index("c")
  sid = lax.axis_index("s")
  w = cid * NS + sid
  r0 = sid * ROWS_PER_TILE
  tbl_cp = pltpu.async_copy(hp_hbm.at[pl.ds(r0, ROWS_PER_TILE)],
                            table_sh.at[pl.ds(r0, ROWS_PER_TILE)], sg.at[0])
  zro_cp = pltpu.async_copy(zeros_hbm.at[pl.ds(r0, ROWS_PER_TILE)],
                            acc_sh.at[pl.ds(r0, ROWS_PER_TILE)], sg.at[1])
  src_cp = pltpu.async_copy(srcm_hbm.at[pl.ds(w * CH_PER_W, CH_PER_W)],
                            src2_v, ss.at[0])
  dst_cp = pltpu.async_copy(dstm_hbm.at[pl.ds(w * CH_PER_W, CH_PER_W)],
                            dst2_v, ss.at[1])
  tbl_cp.wait()
  zro_cp.wait()
  src_cp.wait()
  dst_cp.wait()
  plsc.subcore_barrier()

  def gather(c, i):
    pltpu.async_copy(table_sh.at[src2_v.at[c]], rows_v.at[i], sg.at[i])

  def wait_gather(c, i):
    pltpu.make_async_copy(table_sh.at[src2_v.at[c]], rows_v.at[i],
                          sg.at[i]).wait()

  def scatter(c, i):
    pltpu.async_copy(rows_v.at[i], acc_sh.at[dst2_v.at[c]], ss.at[i], add=True)

  def wait_scatter(c, i):
    pltpu.make_async_copy(rows_v.at[i], acc_sh.at[dst2_v.at[c]],
                          ss.at[i]).wait()

  nfull = (CH_PER_W // depth) * depth
  tail = CH_PER_W - nfull

  for i in range(depth):
    gather(i, i)

  @pl.loop(0, nfull, step=depth)
  def _(k):
    for i in range(depth):
      wait_gather(k + i, i)
      scatter(k + i, i)
    for i in range(depth):
      wait_scatter(k + i, i)
      knext = jnp.where(k + depth + i >= CH_PER_W, i, k + depth + i)
      gather(knext, i)

  for i in range(tail):
    wait_gather(nfull + i, i)
    scatter(nfull + i, i)
  for i in range(tail):
    wait_scatter(nfull + i, i)
  for i in range(tail, depth):
    wait_gather(i, i)

  plsc.subcore_barrier()
  pltpu.sync_copy(acc_sh.at[pl.ds(r0, ROWS_PER_TILE)],
                  out_hbm.at[cid, pl.ds(r0, ROWS_PER_TILE)])


def _agg(hp, srcm, dstm, zeros_np, f, depth):
  mesh = plsc.VectorSubcoreMesh(**_MESH)
  return pl.kernel(
      functools.partial(_agg_body, depth),
      out_type=jax.ShapeDtypeStruct((NC, NP, f), jnp.float32),
      mesh=mesh,
      scratch_types=[
          pltpu.VMEM((CH_PER_W, CHUNK), jnp.int32),
          pltpu.VMEM((CH_PER_W, CHUNK), jnp.int32),
          pltpu.VMEM((depth, CHUNK, f), jnp.float32),
          pltpu.VMEM_SHARED((NP, f), jnp.float32),
          pltpu.VMEM_SHARED((NP, f), jnp.float32),
          pltpu.SemaphoreType.DMA((depth,)),
          pltpu.SemaphoreType.DMA((depth,)),
      ],
      compiler_params=pltpu.CompilerParams(use_tc_tiling_on_sc=False),
  )(hp, srcm, dstm, zeros_np)


def _mm1_body(x_ref, w1_ref, h1_ref):
  h1_ref[...] = jnp.dot(x_ref[...], w1_ref[...],
                        preferred_element_type=jnp.float32,
                        precision=lax.Precision.HIGHEST)


def _mm1(x, W1):
  bm = 1000
  return pl.pallas_call(
      _mm1_body,
      grid=(N // bm,),
      in_specs=[
          pl.BlockSpec((bm, D_IN), lambda i: (i, 0)),
          pl.BlockSpec((D_IN, NHID), lambda i: (0, 0)),
      ],
      out_specs=pl.BlockSpec((bm, NHID), lambda i: (i, 0)),
      out_shape=jax.ShapeDtypeStruct((NP, NHID), jnp.float32),
  )(x, W1)


def _dinv_body(degp_ref, h1_ref, dinv_ref, h1p_ref):
  deg = jnp.sum(degp_ref[...], axis=0) + 1.0
  dinv = lax.rsqrt(deg)[:, None]
  dinv_ref[...] = dinv
  h1p_ref[...] = h1_ref[...] * dinv


def _dinv(degp, h1):
  return pl.pallas_call(
      _dinv_body,
      grid=(1,),
      in_specs=[
          pl.BlockSpec((NW, NP), lambda i: (0, 0)),
          pl.BlockSpec((NP, NHID), lambda i: (0, 0)),
      ],
      out_specs=[
          pl.BlockSpec((NP, 1), lambda i: (0, 0)),
          pl.BlockSpec((NP, NHID), lambda i: (0, 0)),
      ],
      out_shape=[
          jax.ShapeDtypeStruct((NP, 1), jnp.float32),
          jax.ShapeDtypeStruct((NP, NHID), jnp.float32),
      ],
  )(degp, h1)


def _mid_body(p_ref, h1p_ref, dinv_ref, b1_ref, w2_ref, h2p_ref):
  dinv = dinv_ref[...]
  s = (p_ref[0] + p_ref[1] + h1p_ref[...]) * dinv + b1_ref[...]
  h = jnp.maximum(s, 0.0)
  h2 = jnp.dot(h, w2_ref[...], preferred_element_type=jnp.float32,
               precision=lax.Precision.HIGHEST)
  h2p_ref[...] = h2 * dinv


def _mid(p, h1p, dinv, b1, W2):
  bm = 1000
  return pl.pallas_call(
      _mid_body,
      grid=(N // bm,),
      in_specs=[
          pl.BlockSpec((NC, bm, NHID), lambda i: (0, i, 0)),
          pl.BlockSpec((bm, NHID), lambda i: (i, 0)),
          pl.BlockSpec((bm, 1), lambda i: (i, 0)),
          pl.BlockSpec((1, NHID), lambda i: (0, 0)),
          pl.BlockSpec((NHID, NLAT), lambda i: (0, 0)),
      ],
      out_specs=pl.BlockSpec((bm, NLAT), lambda i: (i, 0)),
      out_shape=jax.ShapeDtypeStruct((NP, NLAT), jnp.float32),
  )(p, h1p, dinv, b1, W2)


def _zk_body(q_ref, h2p_ref, dinv_ref, b2_ref, z_ref):
  z = (q_ref[0] + q_ref[1] + h2p_ref[...]) * dinv_ref[...] + b2_ref[...]
  z_ref[...] = z.astype(jnp.bfloat16)


def _zk(q, h2p, dinv, b2):
  bm = 1000
  return pl.pallas_call(
      _zk_body,
      grid=(N // bm,),
      in_specs=[
          pl.BlockSpec((NC, bm, NLAT), lambda i: (0, i, 0)),
          pl.BlockSpec((bm, NLAT), lambda i: (i, 0)),
          pl.BlockSpec((bm, 1), lambda i: (i, 0)),
          pl.BlockSpec((1, NLAT), lambda i: (0, 0)),
      ],
      out_specs=pl.BlockSpec((bm, NLAT), lambda i: (i, 0)),
      out_shape=jax.ShapeDtypeStruct((N, NLAT), jnp.bfloat16),
  )(q, h2p, dinv, b2)


def _dec_body(zi_ref, zj_ref, out_ref):
  out_ref[...] = lax.dot_general(
      zi_ref[...], zj_ref[...],
      dimension_numbers=(((1,), (1,)), ((), ())),
      preferred_element_type=jnp.float32)


def _dec(z):
  bm = 400
  return pl.pallas_call(
      _dec_body,
      grid=(N // bm,),
      in_specs=[
          pl.BlockSpec((bm, NLAT), lambda i: (i, 0)),
          pl.BlockSpec((N, NLAT), lambda i: (0, 0)),
      ],
      out_specs=pl.BlockSpec((bm, N), lambda i: (i, 0)),
      out_shape=jax.ShapeDtypeStruct((N, N), jnp.float32),
      compiler_params=pltpu.CompilerParams(
          dimension_semantics=("parallel",)),
  )(z, z)


def kernel(x, edge_index, W1, b1, W2, b2):
  src = edge_index[0]
  dst = edge_index[1]
  pad = EP - E
  srcp = jnp.concatenate([src, jnp.zeros((pad,), jnp.int32)])
  dstp = jnp.concatenate([dst, jnp.full((pad,), N, jnp.int32)])
  srcm = srcp.reshape(NW * CH_PER_W, CHUNK)
  dstm = dstp.reshape(NW * CH_PER_W, CHUNK)
  zeros64 = jnp.zeros((NP, NHID), jnp.float32)
  zeros16 = jnp.zeros((NP, NLAT), jnp.float32)

  degp = _deg(dstp)
  h1 = _mm1(x, W1)
  dinv, h1p = _dinv(degp, h1)
  p = _agg(h1p, srcm, dstm, zeros64, NHID, 2)
  h2p = _mid(p, h1p, dinv, b1.reshape(1, NHID), W2)
  q = _agg(h2p, srcm, dstm, zeros16, NLAT, 2)
  z = _zk(q, h2p, dinv, b2.reshape(1, NLAT))
  return _dec(z)
